# Initial kernel scaffold; baseline (speedup 1.0000x reference)
import jax
import jax.numpy as jnp
from jax.experimental import pallas as pl


def kernel(feat, proposals, scores, fc1_w, fc1_b, fc2_w, fc2_b, cls_w, cls_b, reg_w, reg_b):
    raise NotImplementedError("write your pallas kernel here")


# speedup vs baseline: 6.2734x; 6.2734x over previous
import jax
import jax.numpy as jnp
from jax import lax
from jax.experimental import pallas as pl
from jax.experimental.pallas import tpu as pltpu

IMG = 800.0
STRIDE = 16
PRE = 6000
POST = 300
OUT = 7
SR = 2
NMS_THR = 0.7
MIN_SIZE = 16.0
NCLS = 21
NEG = -1e30

NPAD = 6144
NROW = 48
NCOL = 128
MROW = 304
HW = 2500
HWPAD = 2504
C = 512
D = C * OUT * OUT
FC = 4096


def _nms_kernel(x1_ref, y1_ref, x2_ref, y2_ref, sc_ref, keep_ref):
    X1 = x1_ref[...]
    Y1 = y1_ref[...]
    X2 = x2_ref[...]
    Y2 = y2_ref[...]
    flat = (lax.broadcasted_iota(jnp.int32, (NROW, NCOL), 0) * NCOL
            + lax.broadcasted_iota(jnp.int32, (NROW, NCOL), 1))
    areas = (X2 - X1) * (Y2 - Y1)
    kiota = lax.broadcasted_iota(jnp.int32, (1, MROW), 1)

    def body(i, carry):
        sc, keep = carry
        m = jnp.max(sc)
        cand = jnp.where(sc == m, flat, jnp.int32(2 ** 30))
        best = jnp.min(cand)
        bm = flat == best
        bx1 = jnp.sum(jnp.where(bm, X1, 0.0))
        by1 = jnp.sum(jnp.where(bm, Y1, 0.0))
        bx2 = jnp.sum(jnp.where(bm, X2, 0.0))
        by2 = jnp.sum(jnp.where(bm, Y2, 0.0))
        xx1 = jnp.maximum(bx1, X1)
        yy1 = jnp.maximum(by1, Y1)
        xx2 = jnp.minimum(bx2, X2)
        yy2 = jnp.minimum(by2, Y2)
        inter = jnp.maximum(xx2 - xx1, 0.0) * jnp.maximum(yy2 - yy1, 0.0)
        a = (bx2 - bx1) * (by2 - by1)
        iou = inter / (a + areas - inter + 1e-9)
        sc = jnp.where((iou > NMS_THR) | bm, NEG, sc)
        keep = jnp.where(kiota == i, best, keep)
        return sc, keep

    _, keep = lax.fori_loop(
        0, POST, body, (sc_ref[...], jnp.zeros((1, MROW), jnp.int32)))
    keep_ref[...] = keep


def _run_nms(txyxy, tsc):
    def padf(v, fill):
        return jnp.concatenate(
            [v, jnp.full((NPAD - PRE,), fill, jnp.float32)]).reshape(NROW, NCOL)

    keep = pl.pallas_call(
        _nms_kernel,
        out_shape=jax.ShapeDtypeStruct((1, MROW), jnp.int32),
    )(padf(txyxy[:, 0], 0.0), padf(txyxy[:, 1], 0.0),
      padf(txyxy[:, 2], 0.0), padf(txyxy[:, 3], 0.0), padf(tsc, NEG))
    return keep[0, :POST]


def _roi_kernel(feat_ref, roi_ref, out_ref):
    scale = 1.0 / STRIDE
    x1 = roi_ref[0, 0, 0] * scale
    y1 = roi_ref[0, 0, 1] * scale
    x2 = roi_ref[0, 0, 2] * scale
    y2 = roi_ref[0, 0, 3] * scale
    bw = (x2 - x1) / OUT
    bh = (y2 - y1) / OUT
    for p in range(OUT * OUT):
        oy = p // OUT
        ox = p % OUT
        acc = jnp.zeros((1, C), jnp.float32)
        for sy in range(SR):
            yg = y1 + (oy + (sy + 0.5) / SR) * bh
            y0f = jnp.floor(yg)
            wy = yg - y0f
            y0i = jnp.clip(y0f.astype(jnp.int32), 0, 49)
            y1i = jnp.minimum(y0i + 1, 49)
            for sx in range(SR):
                xg = x1 + (ox + (sx + 0.5) / SR) * bw
                x0f = jnp.floor(xg)
                wx = xg - x0f
                x0i = jnp.clip(x0f.astype(jnp.int32), 0, 49)
                x1i = jnp.minimum(x0i + 1, 49)
                v00 = feat_ref[pl.ds(y0i * 50 + x0i, 1), :]
                v01 = feat_ref[pl.ds(y0i * 50 + x1i, 1), :]
                v10 = feat_ref[pl.ds(y1i * 50 + x0i, 1), :]
                v11 = feat_ref[pl.ds(y1i * 50 + x1i, 1), :]
                acc = acc + (v00 * ((1 - wy) * (1 - wx)) + v01 * ((1 - wy) * wx)
                             + v10 * (wy * (1 - wx)) + v11 * (wy * wx))
        out_ref[0, pl.ds(p, 1), :] = acc * (1.0 / (SR * SR))


def _run_roi_align(feat2d, rois):
    rois3d = rois.reshape(POST, 1, 4)
    return pl.pallas_call(
        _roi_kernel,
        grid=(POST,),
        in_specs=[
            pl.BlockSpec((HWPAD, C), lambda i: (0, 0)),
            pl.BlockSpec((1, 1, 4), lambda i: (i, 0, 0)),
        ],
        out_specs=pl.BlockSpec((1, OUT * OUT, C), lambda i: (i, 0, 0)),
        out_shape=jax.ShapeDtypeStruct((POST, OUT * OUT, C), jnp.float32),
        compiler_params=pltpu.CompilerParams(
            dimension_semantics=("parallel",)),
    )(feat2d, rois3d)


FC1_NB = 4
FC1_KB = 7
FC1_NBLK = FC // FC1_NB
FC1_KBLK = D // FC1_KB


def _fc1_kernel(x_ref, w_ref, b_ref, o_ref):
    k = pl.program_id(1)
    contrib = jnp.dot(x_ref[...], w_ref[...],
                      preferred_element_type=jnp.float32)

    @pl.when(k == 0)
    def _():
        o_ref[...] = contrib

    @pl.when(k > 0)
    def _():
        o_ref[...] += contrib

    @pl.when(k == FC1_KB - 1)
    def _():
        o_ref[...] = jnp.maximum(o_ref[...] + b_ref[...], 0.0)


def _run_fc1(x, w, b):
    return pl.pallas_call(
        _fc1_kernel,
        grid=(FC1_NB, FC1_KB),
        in_specs=[
            pl.BlockSpec((MROW, FC1_KBLK), lambda n, k: (0, k)),
            pl.BlockSpec((FC1_KBLK, FC1_NBLK), lambda n, k: (k, n)),
            pl.BlockSpec((1, FC1_NBLK), lambda n, k: (0, n)),
        ],
        out_specs=pl.BlockSpec((MROW, FC1_NBLK), lambda n, k: (0, n)),
        out_shape=jax.ShapeDtypeStruct((MROW, FC), jnp.float32),
        compiler_params=pltpu.CompilerParams(
            dimension_semantics=("parallel", "arbitrary")),
    )(x, w, b.reshape(1, FC))


FC2_NB = 4
FC2_NBLK = FC // FC2_NB


def _fc2_kernel(x_ref, w_ref, b_ref, o_ref):
    o_ref[...] = jnp.maximum(
        jnp.dot(x_ref[...], w_ref[...], preferred_element_type=jnp.float32)
        + b_ref[...], 0.0)


def _run_fc2(x, w, b):
    return pl.pallas_call(
        _fc2_kernel,
        grid=(FC2_NB,),
        in_specs=[
            pl.BlockSpec((MROW, FC), lambda n: (0, 0)),
            pl.BlockSpec((FC, FC2_NBLK), lambda n: (0, n)),
            pl.BlockSpec((1, FC2_NBLK), lambda n: (0, n)),
        ],
        out_specs=pl.BlockSpec((MROW, FC2_NBLK), lambda n: (0, n)),
        out_shape=jax.ShapeDtypeStruct((MROW, FC), jnp.float32),
        compiler_params=pltpu.CompilerParams(
            dimension_semantics=("parallel",)),
    )(x, w, b.reshape(1, FC))


def _head_kernel(x_ref, wc_ref, bc_ref, wr_ref, br_ref,
                 sx_ref, sy_ref, sw_ref, sh_ref, cls_ref, dec_ref):
    h = x_ref[...]
    zc = jnp.dot(h, wc_ref[...], preferred_element_type=jnp.float32) + bc_ref[...]
    zr = jnp.dot(h, wr_ref[...], preferred_element_type=jnp.float32) + br_ref[...]
    m = jnp.max(zc, axis=-1, keepdims=True)
    e = jnp.exp(zc - m)
    cls_ref[...] = e / jnp.sum(e, axis=-1, keepdims=True)
    t4 = lax.broadcasted_iota(jnp.int32, (MROW, 4 * NCLS), 1) % 4
    tm2 = t4 % 2
    std = jnp.where(t4 < 2, 0.1, 0.2)
    r = zr * std
    pxy = jnp.where(tm2 == 0, sx_ref[...], sy_ref[...])
    pwh = jnp.where(tm2 == 0, sw_ref[...], sh_ref[...])
    ii = lax.broadcasted_iota(jnp.int32, (4 * NCLS, 4 * NCLS), 0)
    jj = lax.broadcasted_iota(jnp.int32, (4 * NCLS, 4 * NCLS), 1)
    P = jnp.where((jj == ii + 2) & (ii % 4 < 2), 1.0, 0.0)
    rsh = jnp.dot(r, P, preferred_element_type=jnp.float32)
    xy1 = pxy + r * pwh
    xy2 = pxy + rsh * pwh + pwh * jnp.exp(r)
    dec_ref[...] = jnp.clip(jnp.where(t4 < 2, xy1, xy2), 0.0, IMG)


def _run_heads(h2, cls_w, cls_b, reg_w, reg_b, sb):
    sbp = jnp.concatenate(
        [sb, jnp.zeros((MROW - POST, 4), jnp.float32)], axis=0)
    cls, dec = pl.pallas_call(
        _head_kernel,
        out_shape=(jax.ShapeDtypeStruct((MROW, NCLS), jnp.float32),
                   jax.ShapeDtypeStruct((MROW, 4 * NCLS), jnp.float32)),
    )(h2, cls_w, cls_b.reshape(1, NCLS), reg_w, reg_b.reshape(1, 4 * NCLS),
      sbp[:, 0:1], sbp[:, 1:2], sbp[:, 2:3], sbp[:, 3:4])
    return cls[:POST], dec[:POST]


def _per_image(fb, pr, sc, fc1_w, fc1_b, fc2_w, fc2_b,
               cls_w, cls_b, reg_w, reg_b):
    sc = jnp.where((pr[:, 2] < MIN_SIZE) | (pr[:, 3] < MIN_SIZE), NEG, sc)
    tsc, tid = lax.top_k(sc, PRE)
    tb = pr[tid]
    txyxy = jnp.concatenate([tb[:, :2], tb[:, :2] + tb[:, 2:]], -1)
    keep = _run_nms(txyxy, tsc)
    sb = tb[keep]
    sxyxy = txyxy[keep]

    feat2d = jnp.concatenate(
        [fb.transpose(1, 2, 0).reshape(HW, C),
         jnp.zeros((HWPAD - HW, C), jnp.float32)], axis=0)
    pooled = _run_roi_align(feat2d, sxyxy)
    x = pooled.transpose(0, 2, 1).reshape(POST, D)
    x = jnp.concatenate([x, jnp.zeros((MROW - POST, D), jnp.float32)], axis=0)

    h1 = _run_fc1(x, fc1_w, fc1_b)
    h2 = _run_fc2(h1, fc2_w, fc2_b)
    cls, dec = _run_heads(h2, cls_w, cls_b, reg_w, reg_b, sb)
    return jnp.concatenate([cls, dec], -1)


def kernel(feat, proposals, scores, fc1_w, fc1_b, fc2_w, fc2_b,
           cls_w, cls_b, reg_w, reg_b):
    outs = [
        _per_image(feat[b], proposals[b], scores[b], fc1_w, fc1_b,
                   fc2_w, fc2_b, cls_w, cls_b, reg_w, reg_b)
        for b in range(feat.shape[0])
    ]
    return jnp.stack(outs, axis=0)


# speedup vs baseline: 6.4442x; 1.0272x over previous
import jax
import jax.numpy as jnp
from jax import lax
from jax.experimental import pallas as pl
from jax.experimental.pallas import tpu as pltpu

IMG = 800.0
STRIDE = 16
PRE = 6000
POST = 300
OUT = 7
SR = 2
NMS_THR = 0.7
MIN_SIZE = 16.0
NCLS = 21
NEG = -1e30

NPAD = 6144
NROW = 48
NCOL = 128
MROW = 304
HW = 2500
HWPAD = 2504
C = 512
D = C * OUT * OUT
FC = 4096


def _nms_kernel(x1_ref, y1_ref, x2_ref, y2_ref, sc_ref, keep_ref):
    X1 = x1_ref[...]
    Y1 = y1_ref[...]
    X2 = x2_ref[...]
    Y2 = y2_ref[...]
    flat = (lax.broadcasted_iota(jnp.int32, (NROW, NCOL), 0) * NCOL
            + lax.broadcasted_iota(jnp.int32, (NROW, NCOL), 1))
    areas = (X2 - X1) * (Y2 - Y1)
    kiota = lax.broadcasted_iota(jnp.int32, (1, MROW), 1)

    def body(i, carry):
        sc, keep = carry
        m = jnp.max(sc)
        cand = jnp.where(sc == m, flat, jnp.int32(2 ** 30))
        best = jnp.min(cand)
        bm = flat == best
        bx1 = jnp.sum(jnp.where(bm, X1, 0.0))
        by1 = jnp.sum(jnp.where(bm, Y1, 0.0))
        bx2 = jnp.sum(jnp.where(bm, X2, 0.0))
        by2 = jnp.sum(jnp.where(bm, Y2, 0.0))
        xx1 = jnp.maximum(bx1, X1)
        yy1 = jnp.maximum(by1, Y1)
        xx2 = jnp.minimum(bx2, X2)
        yy2 = jnp.minimum(by2, Y2)
        inter = jnp.maximum(xx2 - xx1, 0.0) * jnp.maximum(yy2 - yy1, 0.0)
        a = (bx2 - bx1) * (by2 - by1)
        iou = inter / (a + areas - inter + 1e-9)
        sc = jnp.where((iou > NMS_THR) | bm, NEG, sc)
        keep = jnp.where(kiota == i, best, keep)
        return sc, keep

    _, keep = lax.fori_loop(
        0, POST, body, (sc_ref[...], jnp.zeros((1, MROW), jnp.int32)))
    keep_ref[...] = keep


def _run_nms(txyxy, tsc):
    def padf(v, fill):
        return jnp.concatenate(
            [v, jnp.full((NPAD - PRE,), fill, jnp.float32)]).reshape(NROW, NCOL)

    keep = pl.pallas_call(
        _nms_kernel,
        out_shape=jax.ShapeDtypeStruct((1, MROW), jnp.int32),
    )(padf(txyxy[:, 0], 0.0), padf(txyxy[:, 1], 0.0),
      padf(txyxy[:, 2], 0.0), padf(txyxy[:, 3], 0.0), padf(tsc, NEG))
    return keep[0, :POST]


ROIPG = 4


def _roi_kernel(feat_ref, roi_ref, out_ref):
    scale = 1.0 / STRIDE
    for r in range(ROIPG):
        x1 = roi_ref[0, r, 0] * scale
        y1 = roi_ref[0, r, 1] * scale
        x2 = roi_ref[0, r, 2] * scale
        y2 = roi_ref[0, r, 3] * scale
        bw = (x2 - x1) / OUT
        bh = (y2 - y1) / OUT
        for p in range(OUT * OUT):
            oy = p // OUT
            ox = p % OUT
            acc = jnp.zeros((1, C), jnp.float32)
            for sy in range(SR):
                yg = y1 + (oy + (sy + 0.5) / SR) * bh
                y0f = jnp.floor(yg)
                wy = yg - y0f
                y0i = jnp.clip(y0f.astype(jnp.int32), 0, 49)
                y1i = jnp.minimum(y0i + 1, 49)
                for sx in range(SR):
                    xg = x1 + (ox + (sx + 0.5) / SR) * bw
                    x0f = jnp.floor(xg)
                    wx = xg - x0f
                    x0i = jnp.clip(x0f.astype(jnp.int32), 0, 49)
                    x1i = jnp.minimum(x0i + 1, 49)
                    v00 = feat_ref[pl.ds(y0i * 50 + x0i, 1), :]
                    v01 = feat_ref[pl.ds(y0i * 50 + x1i, 1), :]
                    v10 = feat_ref[pl.ds(y1i * 50 + x0i, 1), :]
                    v11 = feat_ref[pl.ds(y1i * 50 + x1i, 1), :]
                    acc = acc + (v00 * ((1 - wy) * (1 - wx))
                                 + v01 * ((1 - wy) * wx)
                                 + v10 * (wy * (1 - wx)) + v11 * (wy * wx))
            out_ref[r, pl.ds(p, 1), :] = acc * (1.0 / (SR * SR))


def _run_roi_align(feat2d, rois):
    rois3d = rois.reshape(POST // ROIPG, ROIPG, 4)
    return pl.pallas_call(
        _roi_kernel,
        grid=(POST // ROIPG,),
        in_specs=[
            pl.BlockSpec((HWPAD, C), lambda i: (0, 0)),
            pl.BlockSpec((1, ROIPG, 4), lambda i: (i, 0, 0)),
        ],
        out_specs=pl.BlockSpec((ROIPG, OUT * OUT, C), lambda i: (i, 0, 0)),
        out_shape=jax.ShapeDtypeStruct((POST, OUT * OUT, C), jnp.float32),
        compiler_params=pltpu.CompilerParams(
            dimension_semantics=("parallel",)),
    )(feat2d, rois3d)


FC1_NB = 4
FC1_KB = 7
FC1_NBLK = FC // FC1_NB
FC1_KBLK = D // FC1_KB


def _fc1_kernel(x_ref, w_ref, b_ref, o_ref):
    k = pl.program_id(1)
    contrib = jnp.dot(x_ref[...], w_ref[...],
                      preferred_element_type=jnp.float32)

    @pl.when(k == 0)
    def _():
        o_ref[...] = contrib

    @pl.when(k > 0)
    def _():
        o_ref[...] += contrib

    @pl.when(k == FC1_KB - 1)
    def _():
        o_ref[...] = jnp.maximum(o_ref[...] + b_ref[...], 0.0)


def _run_fc1(x, w, b):
    return pl.pallas_call(
        _fc1_kernel,
        grid=(FC1_NB, FC1_KB),
        in_specs=[
            pl.BlockSpec((MROW, FC1_KBLK), lambda n, k: (0, k)),
            pl.BlockSpec((FC1_KBLK, FC1_NBLK), lambda n, k: (k, n)),
            pl.BlockSpec((1, FC1_NBLK), lambda n, k: (0, n)),
        ],
        out_specs=pl.BlockSpec((MROW, FC1_NBLK), lambda n, k: (0, n)),
        out_shape=jax.ShapeDtypeStruct((MROW, FC), jnp.float32),
        compiler_params=pltpu.CompilerParams(
            dimension_semantics=("parallel", "arbitrary")),
    )(x, w, b.reshape(1, FC))


FC2_NB = 4
FC2_NBLK = FC // FC2_NB


def _fc2_kernel(x_ref, w_ref, b_ref, o_ref):
    o_ref[...] = jnp.maximum(
        jnp.dot(x_ref[...], w_ref[...], preferred_element_type=jnp.float32)
        + b_ref[...], 0.0)


def _run_fc2(x, w, b):
    return pl.pallas_call(
        _fc2_kernel,
        grid=(FC2_NB,),
        in_specs=[
            pl.BlockSpec((MROW, FC), lambda n: (0, 0)),
            pl.BlockSpec((FC, FC2_NBLK), lambda n: (0, n)),
            pl.BlockSpec((1, FC2_NBLK), lambda n: (0, n)),
        ],
        out_specs=pl.BlockSpec((MROW, FC2_NBLK), lambda n: (0, n)),
        out_shape=jax.ShapeDtypeStruct((MROW, FC), jnp.float32),
        compiler_params=pltpu.CompilerParams(
            dimension_semantics=("parallel",)),
    )(x, w, b.reshape(1, FC))


def _head_kernel(x_ref, wc_ref, bc_ref, wr_ref, br_ref,
                 sx_ref, sy_ref, sw_ref, sh_ref, cls_ref, dec_ref):
    h = x_ref[...]
    zc = jnp.dot(h, wc_ref[...], preferred_element_type=jnp.float32) + bc_ref[...]
    zr = jnp.dot(h, wr_ref[...], preferred_element_type=jnp.float32) + br_ref[...]
    m = jnp.max(zc, axis=-1, keepdims=True)
    e = jnp.exp(zc - m)
    cls_ref[...] = e / jnp.sum(e, axis=-1, keepdims=True)
    t4 = lax.broadcasted_iota(jnp.int32, (MROW, 4 * NCLS), 1) % 4
    tm2 = t4 % 2
    std = jnp.where(t4 < 2, 0.1, 0.2)
    r = zr * std
    pxy = jnp.where(tm2 == 0, sx_ref[...], sy_ref[...])
    pwh = jnp.where(tm2 == 0, sw_ref[...], sh_ref[...])
    ii = lax.broadcasted_iota(jnp.int32, (4 * NCLS, 4 * NCLS), 0)
    jj = lax.broadcasted_iota(jnp.int32, (4 * NCLS, 4 * NCLS), 1)
    P = jnp.where((jj == ii + 2) & (ii % 4 < 2), 1.0, 0.0)
    rsh = jnp.dot(r, P, preferred_element_type=jnp.float32)
    xy1 = pxy + r * pwh
    xy2 = pxy + rsh * pwh + pwh * jnp.exp(r)
    dec_ref[...] = jnp.clip(jnp.where(t4 < 2, xy1, xy2), 0.0, IMG)


def _run_heads(h2, cls_w, cls_b, reg_w, reg_b, sb):
    sbp = jnp.concatenate(
        [sb, jnp.zeros((MROW - POST, 4), jnp.float32)], axis=0)
    cls, dec = pl.pallas_call(
        _head_kernel,
        out_shape=(jax.ShapeDtypeStruct((MROW, NCLS), jnp.float32),
                   jax.ShapeDtypeStruct((MROW, 4 * NCLS), jnp.float32)),
    )(h2, cls_w, cls_b.reshape(1, NCLS), reg_w, reg_b.reshape(1, 4 * NCLS),
      sbp[:, 0:1], sbp[:, 1:2], sbp[:, 2:3], sbp[:, 3:4])
    return cls[:POST], dec[:POST]


def _per_image(fb, pr, sc, fc1_w, fc1_b, fc2_w, fc2_b,
               cls_w, cls_b, reg_w, reg_b):
    sc = jnp.where((pr[:, 2] < MIN_SIZE) | (pr[:, 3] < MIN_SIZE), NEG, sc)
    tsc, tid = lax.top_k(sc, PRE)
    tb = pr[tid]
    txyxy = jnp.concatenate([tb[:, :2], tb[:, :2] + tb[:, 2:]], -1)
    keep = _run_nms(txyxy, tsc)
    sb = tb[keep]
    sxyxy = txyxy[keep]

    feat2d = jnp.concatenate(
        [fb.transpose(1, 2, 0).reshape(HW, C),
         jnp.zeros((HWPAD - HW, C), jnp.float32)], axis=0)
    pooled = _run_roi_align(feat2d, sxyxy)
    x = pooled.transpose(0, 2, 1).reshape(POST, D)
    x = jnp.concatenate([x, jnp.zeros((MROW - POST, D), jnp.float32)], axis=0)

    h1 = _run_fc1(x, fc1_w, fc1_b)
    h2 = _run_fc2(h1, fc2_w, fc2_b)
    cls, dec = _run_heads(h2, cls_w, cls_b, reg_w, reg_b, sb)
    return jnp.concatenate([cls, dec], -1)


def kernel(feat, proposals, scores, fc1_w, fc1_b, fc2_w, fc2_b,
           cls_w, cls_b, reg_w, reg_b):
    outs = [
        _per_image(feat[b], proposals[b], scores[b], fc1_w, fc1_b,
                   fc2_w, fc2_b, cls_w, cls_b, reg_w, reg_b)
        for b in range(feat.shape[0])
    ]
    return jnp.stack(outs, axis=0)
